# baseline (speedup 1.0000x reference)
import functools

import jax
import jax.numpy as jnp
from jax import lax
from jax.experimental import pallas as pl
from jax.experimental.pallas import tpu as pltpu
from jax.experimental.pallas import tpu_sc as plsc

_LANES = 16
_NUM_CORES = 2
_L = 3


def _gather_tile_kernel(B, D, IDX_PAD):
    mesh = plsc.VectorSubcoreMesh(core_axis_name="c", subcore_axis_name="s")

    @functools.partial(
        pl.kernel,
        mesh=mesh,
        out_type=jax.ShapeDtypeStruct((B, _L, D), jnp.float32),
        scratch_types=[
            pltpu.VMEM((IDX_PAD,), jnp.int32),
            pltpu.VMEM((IDX_PAD, D), jnp.float32),
            pltpu.SemaphoreType.DMA,
        ],
    )
    def gather_tile(x_hbm, idx_hbm, out_hbm, idx_v, rows_v, sem):
        wid = lax.axis_index("s") * _NUM_CORES + lax.axis_index("c")

        @pl.when(wid < _L)
        def _():
            pltpu.sync_copy(idx_hbm, idx_v)
            for i in range(IDX_PAD // _LANES):
                sl = pl.ds(i * _LANES, _LANES)
                idx_v[sl] = idx_v[sl] + 1
            pltpu.async_copy(x_hbm.at[idx_v], rows_v, sem).wait()
            pltpu.sync_copy(rows_v.at[pl.ds(0, B)], out_hbm.at[:, wid, :])

    return gather_tile


def kernel(node_feat, node_id, edge_type, norm, x_input, rel_labels, W1, b1):
    B = rel_labels.shape[0]
    D = x_input.shape[1]
    idx_pad = -(-B // _LANES) * _LANES
    idx = jnp.pad(rel_labels.astype(jnp.int32), (0, idx_pad - B))
    out = _gather_tile_kernel(B, D, idx_pad)(x_input, idx)
    return out.reshape(B, _L * D)


# speedup vs baseline: 1.0621x; 1.0621x over previous
import functools

import jax
import jax.numpy as jnp
from jax import lax
from jax.experimental import pallas as pl
from jax.experimental.pallas import tpu as pltpu
from jax.experimental.pallas import tpu_sc as plsc

_LANES = 16
_NUM_CORES = 2
_L = 3


def _gather_tile_kernel(B, D, IDX_PAD):
    mesh = plsc.VectorSubcoreMesh(
        core_axis_name="c", subcore_axis_name="s", num_cores=1
    )

    @functools.partial(
        pl.kernel,
        mesh=mesh,
        out_type=jax.ShapeDtypeStruct((B, _L, D), jnp.float32),
        scratch_types=[
            pltpu.VMEM((IDX_PAD,), jnp.int32),
            pltpu.VMEM((IDX_PAD, D), jnp.float32),
            pltpu.SemaphoreType.DMA,
        ],
    )
    def gather_tile(x_hbm, idx_hbm, out_hbm, idx_v, rows_v, sem):
        wid = lax.axis_index("s")

        @pl.when(wid < _L)
        def _():
            pltpu.sync_copy(idx_hbm, idx_v)
            for i in range(IDX_PAD // _LANES):
                sl = pl.ds(i * _LANES, _LANES)
                idx_v[sl] = idx_v[sl] + 1
            pltpu.async_copy(x_hbm.at[idx_v], rows_v, sem).wait()
            pltpu.sync_copy(rows_v.at[pl.ds(0, B)], out_hbm.at[:, wid, :])

    return gather_tile


def kernel(node_feat, node_id, edge_type, norm, x_input, rel_labels, W1, b1):
    B = rel_labels.shape[0]
    D = x_input.shape[1]
    idx_pad = -(-B // _LANES) * _LANES
    idx = jnp.pad(rel_labels.astype(jnp.int32), (0, idx_pad - B))
    out = _gather_tile_kernel(B, D, idx_pad)(x_input, idx)
    return out.reshape(B, _L * D)


# speedup vs baseline: 1.1059x; 1.0412x over previous
import functools

import jax
import jax.numpy as jnp
from jax import lax
from jax.experimental import pallas as pl
from jax.experimental.pallas import tpu as pltpu
from jax.experimental.pallas import tpu_sc as plsc

_LANES = 16
_NUM_CORES = 2
_L = 3


def _gather_tile_kernel(B, D, IDX_PAD):
    mesh = plsc.VectorSubcoreMesh(
        core_axis_name="c", subcore_axis_name="s", num_cores=1
    )

    @functools.partial(
        pl.kernel,
        mesh=mesh,
        out_type=jax.ShapeDtypeStruct((B, _L, D), jnp.float32),
        scratch_types=[
            pltpu.VMEM((IDX_PAD,), jnp.int32),
            pltpu.VMEM((IDX_PAD, D), jnp.float32),
            pltpu.SemaphoreType.DMA,
        ],
    )
    def gather_tile(x_hbm, idx_hbm, out_hbm, idx_v, rows_v, sem):
        wid = lax.axis_index("s")

        @pl.when(wid < _L)
        def _():
            pltpu.sync_copy(idx_hbm, idx_v)
            pltpu.async_copy(x_hbm.at[idx_v], rows_v, sem).wait()
            pltpu.sync_copy(rows_v.at[pl.ds(0, B)], out_hbm.at[:, wid, :])

    return gather_tile


def kernel(node_feat, node_id, edge_type, norm, x_input, rel_labels, W1, b1):
    B = rel_labels.shape[0]
    D = x_input.shape[1]
    idx_pad = -(-B // _LANES) * _LANES
    idx = jnp.pad(rel_labels.astype(jnp.int32) + 1, (0, idx_pad - B))
    out = _gather_tile_kernel(B, D, idx_pad)(x_input, idx)
    return out.reshape(B, _L * D)


# speedup vs baseline: 1.1711x; 1.0589x over previous
import functools

import jax
import jax.numpy as jnp
from jax import lax
from jax.experimental import pallas as pl
from jax.experimental.pallas import tpu as pltpu
from jax.experimental.pallas import tpu_sc as plsc

_LANES = 16
_NUM_CORES = 2
_L = 3


def _gather_tile_kernel(B, D, IDX_PAD):
    mesh = plsc.VectorSubcoreMesh(
        core_axis_name="c", subcore_axis_name="s", num_cores=1
    )

    @functools.partial(
        pl.kernel,
        mesh=mesh,
        out_type=jax.ShapeDtypeStruct((B, _L, D), jnp.float32),
        scratch_types=[
            pltpu.VMEM((IDX_PAD,), jnp.int32),
            pltpu.VMEM((IDX_PAD, D), jnp.float32),
            pltpu.SemaphoreType.DMA,
        ],
    )
    def gather_tile(x_hbm, idx_hbm, out_hbm, idx_v, rows_v, sem):
        wid = lax.axis_index("s")

        @pl.when(wid < _L)
        def _():
            pltpu.sync_copy(idx_hbm, idx_v)
            pltpu.async_copy(x_hbm.at[idx_v], rows_v, sem).wait()
            pltpu.sync_copy(rows_v, out_hbm.at[:, wid, :])

    return gather_tile


def kernel(node_feat, node_id, edge_type, norm, x_input, rel_labels, W1, b1):
    B = rel_labels.shape[0]
    D = x_input.shape[1]
    idx_pad = B
    idx = rel_labels.astype(jnp.int32) + 1
    out = _gather_tile_kernel(B, D, idx_pad)(x_input, idx)
    return out.reshape(B, _L * D)


# speedup vs baseline: 1.2116x; 1.0346x over previous
import functools

import jax
import jax.numpy as jnp
from jax import lax
from jax.experimental import pallas as pl
from jax.experimental.pallas import tpu as pltpu
from jax.experimental.pallas import tpu_sc as plsc

_LANES = 16
_NUM_CORES = 2
_L = 3


def _gather_tile_kernel(B, D, IDX_PAD):
    mesh = plsc.VectorSubcoreMesh(
        core_axis_name="c", subcore_axis_name="s", num_cores=1
    )

    r_split = (B // 2) & ~7
    splits = ((0, r_split), (r_split, B - r_split))

    @functools.partial(
        pl.kernel,
        mesh=mesh,
        out_type=jax.ShapeDtypeStruct((B, _L, D), jnp.float32),
        scratch_types=[
            pltpu.VMEM((IDX_PAD,), jnp.int32),
            pltpu.VMEM((IDX_PAD, D), jnp.float32),
            pltpu.SemaphoreType.DMA,
        ],
    )
    def gather_tile(x_hbm, idx_hbm, out_hbm, idx_v, rows_v, sem):
        wid = lax.axis_index("s")
        for half, (r0, nr) in enumerate(splits):

            @pl.when((wid >= half * _L) & (wid < (half + 1) * _L))
            def _(r0=r0, nr=nr, half=half):
                l = wid - half * _L
                idx_h = idx_v.at[pl.ds(0, nr)]
                rows_h = rows_v.at[pl.ds(0, nr)]
                pltpu.sync_copy(idx_hbm.at[pl.ds(r0, nr)], idx_h)
                pltpu.async_copy(x_hbm.at[idx_h], rows_h, sem).wait()
                pltpu.sync_copy(rows_h, out_hbm.at[pl.ds(r0, nr), l, :])

    return gather_tile


def kernel(node_feat, node_id, edge_type, norm, x_input, rel_labels, W1, b1):
    B = rel_labels.shape[0]
    D = x_input.shape[1]
    idx_pad = B
    idx = rel_labels.astype(jnp.int32) + 1
    out = _gather_tile_kernel(B, D, idx_pad)(x_input, idx)
    return out.reshape(B, _L * D)
